# baseline (speedup 1.0000x reference)
import functools

import jax
import jax.numpy as jnp
from jax.experimental import pallas as pl
from jax.experimental.pallas import tpu as pltpu

_BLK = 128
_CACHE_BLOCKS = 13


def _s1_kernel(x_ref, W1_ref, s1_ref):
    s1_ref[...] = jnp.dot(x_ref[...], W1_ref[...],
                          preferred_element_type=jnp.float32
                          ).astype(jnp.bfloat16)


def _fused_kernel(s1_ref, adj_ref, W4_ref, b1_ref, b4_ref, out_ref,
                  cache_ref, s4_ref, s4bf_ref, *, n, blk, cache_blocks, nb):
    p = pl.program_id(0)
    i = pl.program_id(1)

    @pl.when(p == 0)
    def _phase0():
        a_bf = adj_ref[...].astype(jnp.bfloat16)
        h = jnp.dot(a_bf, s1_ref[...],
                    preferred_element_type=jnp.float32)
        h = jnp.maximum(h + b1_ref[...], 0.0)
        s4_ref[pl.ds(i * blk, blk), :] = jnp.dot(
            h, W4_ref[...], preferred_element_type=jnp.float32)

        @pl.when(i < cache_blocks)
        def _stash():
            cache_ref[pl.ds(i * blk, blk), :] = a_bf

    @pl.when(jnp.logical_and(p == 1, i == 0))
    def _prep():
        s4bf_ref[...] = s4_ref[...].astype(jnp.bfloat16)

    @pl.when(jnp.logical_and(p == 1, i < cache_blocks))
    def _phase1_cached():
        o = jnp.dot(cache_ref[pl.ds(i * blk, blk), :],
                    s4bf_ref[pl.ds(0, n), :],
                    preferred_element_type=jnp.float32) + b4_ref[...]
        m = jnp.max(o, axis=1, keepdims=True)
        lse = jnp.log(jnp.sum(jnp.exp(o - m), axis=1, keepdims=True)) + m
        out_ref[...] = o - lse

    @pl.when(jnp.logical_and(p == 1, i >= cache_blocks))
    def _phase1_hbm():
        o = jnp.dot(adj_ref[...].astype(jnp.bfloat16),
                    s4bf_ref[pl.ds(0, n), :],
                    preferred_element_type=jnp.float32) + b4_ref[...]
        m = jnp.max(o, axis=1, keepdims=True)
        lse = jnp.log(jnp.sum(jnp.exp(o - m), axis=1, keepdims=True)) + m
        out_ref[...] = o - lse


def kernel(x, adj, W1, b1, W4, b4):
    n, nfeat = x.shape
    nhid = W1.shape[1]
    nclass = W4.shape[1]

    b1_2d = b1.reshape(1, nhid)
    b4_2d = b4.reshape(1, nclass)

    blk = _BLK
    nb = pl.cdiv(n, blk)
    cache_blocks = _CACHE_BLOCKS
    npad = nb * blk

    s1 = pl.pallas_call(
        _s1_kernel,
        in_specs=[
            pl.BlockSpec((n, nfeat), lambda: (0, 0)),
            pl.BlockSpec((nfeat, nhid), lambda: (0, 0)),
        ],
        out_specs=pl.BlockSpec((n, nhid), lambda: (0, 0)),
        out_shape=jax.ShapeDtypeStruct((n, nhid), jnp.bfloat16),
    )(x, W1)

    body = functools.partial(_fused_kernel, n=n, blk=blk,
                             cache_blocks=cache_blocks, nb=nb)
    out = pl.pallas_call(
        body,
        grid=(2, nb),
        in_specs=[
            pl.BlockSpec((n, nhid), lambda p, i: (0, 0)),
            pl.BlockSpec((blk, n),
                         lambda p, i: (jnp.where(p == 0, i,
                                                 jnp.maximum(i, cache_blocks)),
                                       0)),
            pl.BlockSpec((nhid, nclass), lambda p, i: (0, 0)),
            pl.BlockSpec((1, nhid), lambda p, i: (0, 0)),
            pl.BlockSpec((1, nclass), lambda p, i: (0, 0)),
        ],
        out_specs=pl.BlockSpec((blk, nclass), lambda p, i: (i, 0)),
        out_shape=jax.ShapeDtypeStruct((n, nclass), jnp.float32),
        scratch_shapes=[
            pltpu.VMEM((cache_blocks * blk, n), jnp.bfloat16),
            pltpu.VMEM((npad, nclass), jnp.float32),
            pltpu.VMEM((npad, nclass), jnp.bfloat16),
        ],
        compiler_params=pltpu.CompilerParams(
            dimension_semantics=("arbitrary", "arbitrary"),
        ),
    )(s1, adj, W4, b1_2d, b4_2d)
    return out


# speedup vs baseline: 1.1460x; 1.1460x over previous
import jax
import jax.numpy as jnp
from jax.experimental import pallas as pl
from jax.experimental.pallas import tpu as pltpu

_BLK = 512


def _s1_kernel(x_ref, W1_ref, s1_ref):
    s1_ref[...] = jnp.dot(x_ref[...], W1_ref[...],
                          preferred_element_type=jnp.float32
                          ).astype(jnp.bfloat16)


def _fused_kernel(s1_ref, adj_ref, W4_ref, b1_ref, b4_ref, out_ref, s4_ref,
                  *, blk):
    p = pl.program_id(0)
    i = pl.program_id(1)

    @pl.when(p == 0)
    def _phase0():
        h = jnp.dot(adj_ref[...].astype(jnp.bfloat16), s1_ref[...],
                    preferred_element_type=jnp.float32)
        h = jnp.maximum(h + b1_ref[...], 0.0)
        s4_ref[pl.ds(i * blk, blk), :] = jnp.dot(
            h, W4_ref[...], preferred_element_type=jnp.float32
        ).astype(jnp.bfloat16)

    @pl.when(p == 1)
    def _phase1():
        n = adj_ref.shape[1]
        o = jnp.dot(adj_ref[...].astype(jnp.bfloat16),
                    s4_ref[pl.ds(0, n), :],
                    preferred_element_type=jnp.float32) + b4_ref[...]
        m = jnp.max(o, axis=1, keepdims=True)
        lse = jnp.log(jnp.sum(jnp.exp(o - m), axis=1, keepdims=True)) + m
        out_ref[...] = o - lse


def kernel(x, adj, W1, b1, W4, b4):
    n, nfeat = x.shape
    nhid = W1.shape[1]
    nclass = W4.shape[1]

    b1_2d = b1.reshape(1, nhid)
    b4_2d = b4.reshape(1, nclass)

    blk = _BLK if n > _BLK else n
    nb = pl.cdiv(n, blk)
    npad = nb * blk

    s1 = pl.pallas_call(
        _s1_kernel,
        in_specs=[
            pl.BlockSpec((n, nfeat), lambda: (0, 0)),
            pl.BlockSpec((nfeat, nhid), lambda: (0, 0)),
        ],
        out_specs=pl.BlockSpec((n, nhid), lambda: (0, 0)),
        out_shape=jax.ShapeDtypeStruct((n, nhid), jnp.bfloat16),
    )(x, W1)

    import functools
    body = functools.partial(_fused_kernel, blk=blk)
    out = pl.pallas_call(
        body,
        grid=(2, nb),
        in_specs=[
            pl.BlockSpec((n, nhid), lambda p, i: (0, 0)),
            pl.BlockSpec((blk, n), lambda p, i: (i, 0)),
            pl.BlockSpec((nhid, nclass), lambda p, i: (0, 0)),
            pl.BlockSpec((1, nhid), lambda p, i: (0, 0)),
            pl.BlockSpec((1, nclass), lambda p, i: (0, 0)),
        ],
        out_specs=pl.BlockSpec((blk, nclass), lambda p, i: (i, 0)),
        out_shape=jax.ShapeDtypeStruct((n, nclass), jnp.float32),
        scratch_shapes=[
            pltpu.VMEM((npad, nclass), jnp.bfloat16),
        ],
        compiler_params=pltpu.CompilerParams(
            dimension_semantics=("arbitrary", "arbitrary"),
        ),
    )(s1, adj, W4, b1_2d, b4_2d)
    return out


# speedup vs baseline: 1.1891x; 1.0376x over previous
import functools

import jax
import jax.numpy as jnp
from jax.experimental import pallas as pl
from jax.experimental.pallas import tpu as pltpu

_BLK = 400
_CACHE = 2


def _s1_kernel(x_ref, W1_ref, s1_ref):
    s1_ref[...] = jnp.dot(x_ref[...], W1_ref[...],
                          preferred_element_type=jnp.float32
                          ).astype(jnp.bfloat16)


def _fused_kernel(s1_ref, adj_ref, W4_ref, b1_ref, b4_ref, out_ref,
                  s4_ref, cache_ref, *, blk, nb, cache):
    p = pl.program_id(0)
    i = pl.program_id(1)

    def _finish(o):
        m = jnp.max(o, axis=1, keepdims=True)
        lse = jnp.log(jnp.sum(jnp.exp(o - m), axis=1, keepdims=True)) + m
        out_ref[...] = o - lse

    @pl.when(p == 0)
    def _phase0():
        h = jnp.dot(adj_ref[...], s1_ref[...].astype(jnp.float32),
                    preferred_element_type=jnp.float32)
        h = jnp.maximum(h + b1_ref[...], 0.0)
        s4_ref[pl.ds(i * blk, blk), :] = jnp.dot(
            h, W4_ref[...], preferred_element_type=jnp.float32
        ).astype(jnp.bfloat16)

        @pl.when(i < cache)
        def _stash():
            cache_ref[pl.ds(i * blk, blk), :] = (
                adj_ref[...].astype(jnp.bfloat16))

    @pl.when(jnp.logical_and(p == 1, i < nb - cache))
    def _phase1_hbm():
        _finish(jnp.dot(adj_ref[...], s4_ref[...].astype(jnp.float32),
                        preferred_element_type=jnp.float32) + b4_ref[...])

    @pl.when(jnp.logical_and(p == 1, i >= nb - cache))
    def _phase1_cached():
        j = i - (nb - cache)
        _finish(jnp.dot(cache_ref[pl.ds(j * blk, blk), :], s4_ref[...],
                        preferred_element_type=jnp.float32) + b4_ref[...])


def kernel(x, adj, W1, b1, W4, b4):
    n, nfeat = x.shape
    nhid = W1.shape[1]
    nclass = W4.shape[1]

    b1_2d = b1.reshape(1, nhid)
    b4_2d = b4.reshape(1, nclass)

    blk = _BLK if n % _BLK == 0 else n
    nb = n // blk
    cache = _CACHE if nb > _CACHE else 0

    s1 = pl.pallas_call(
        _s1_kernel,
        in_specs=[
            pl.BlockSpec((n, nfeat), lambda: (0, 0)),
            pl.BlockSpec((nfeat, nhid), lambda: (0, 0)),
        ],
        out_specs=pl.BlockSpec((n, nhid), lambda: (0, 0)),
        out_shape=jax.ShapeDtypeStruct((n, nhid), jnp.bfloat16),
    )(x, W1)

    body = functools.partial(_fused_kernel, blk=blk, nb=nb, cache=cache)
    out = pl.pallas_call(
        body,
        grid=(2, nb),
        in_specs=[
            pl.BlockSpec((n, nhid), lambda p, i: (0, 0)),
            pl.BlockSpec(
                (blk, n),
                lambda p, i: (jnp.where(p == 0, i,
                                        jnp.minimum(i + cache, nb - 1)), 0),
            ),
            pl.BlockSpec((nhid, nclass), lambda p, i: (0, 0)),
            pl.BlockSpec((1, nhid), lambda p, i: (0, 0)),
            pl.BlockSpec((1, nclass), lambda p, i: (0, 0)),
        ],
        out_specs=pl.BlockSpec(
            (blk, nclass),
            lambda p, i: (jnp.where(p == 0, i, (i + cache) % nb), 0)),
        out_shape=jax.ShapeDtypeStruct((n, nclass), jnp.float32),
        scratch_shapes=[
            pltpu.VMEM((n, nclass), jnp.bfloat16),
            pltpu.VMEM((max(cache, 1) * blk, n), jnp.bfloat16),
        ],
        compiler_params=pltpu.CompilerParams(
            dimension_semantics=("arbitrary", "arbitrary"),
        ),
    )(s1, adj, W4, b1_2d, b4_2d)
    return out
